# baseline (speedup 1.0000x reference)
import functools
import jax
import jax.numpy as jnp
from jax import lax
from jax.experimental import pallas as pl
from jax.experimental.pallas import tpu as pltpu
from jax.experimental.pallas import tpu_sc as plsc

_B = 4096
_L = 200
_HID = 64
_NC = 2
_NS = 16
_NW = _NC * _NS
_N = _B * _L
_ROWS_W = _N // _NW
_CH = _L
_NCHUNK = _ROWS_W // _CH
_LANES = 16


def _embed_body(x_hbm, pos_hbm, tok_hbm, out_hbm, idx_v, pos_v, buf_v, gsem, osem):
    wid = lax.axis_index("s") * _NC + lax.axis_index("c")
    fbase = wid * _ROWS_W

    pltpu.sync_copy(x_hbm.at[pl.ds(fbase, _ROWS_W)], idx_v)
    pltpu.sync_copy(pos_hbm, pos_v)

    def chunk_body(c, _):
        off = c * _CH
        d0 = pltpu.async_copy(
            tok_hbm.at[idx_v.at[pl.ds(off, 128)]],
            buf_v.at[pl.ds(0, 128)], gsem)
        d1 = pltpu.async_copy(
            tok_hbm.at[idx_v.at[pl.ds(off + 128, _CH - 128)]],
            buf_v.at[pl.ds(128, _CH - 128)], gsem)
        d0.wait()
        d1.wait()

        def row_body(r, _):
            for j in range(_HID // _LANES):
                sl = pl.ds(j * _LANES, _LANES)
                plsc.addupdate(buf_v.at[r, sl], pos_v[r, sl])
            return ()

        lax.fori_loop(0, _CH, row_body, (), unroll=2)

        pltpu.async_copy(buf_v, out_hbm.at[pl.ds(fbase + off, _CH)], osem).wait()
        return ()

    lax.fori_loop(0, _NCHUNK, chunk_body, ())


@jax.jit
def _embed(x_flat, pos_block, tok_table):
    mesh = plsc.VectorSubcoreMesh(core_axis_name="c", subcore_axis_name="s")
    run = functools.partial(
        pl.kernel,
        out_type=jax.ShapeDtypeStruct((_N, _HID), jnp.float32),
        mesh=mesh,
        scratch_types=[
            pltpu.VMEM((_ROWS_W,), jnp.int32),
            pltpu.VMEM((_L, _HID), jnp.float32),
            pltpu.VMEM((_CH, _HID), jnp.float32),
            pltpu.SemaphoreType.DMA,
            pltpu.SemaphoreType.DMA,
        ],
        compiler_params=pltpu.CompilerParams(use_tc_tiling_on_sc=False),
    )(_embed_body)
    return run(x_flat, pos_block, tok_table)


def kernel(x, tok_table, pos_table):
    x_flat = x.reshape(_N).astype(jnp.int32)
    pos_block = pos_table[:_L]
    out = _embed(x_flat, pos_block, tok_table)
    return out.reshape(_B, _L, _HID)
